# Initial kernel scaffold; baseline (speedup 1.0000x reference)
import jax
import jax.numpy as jnp
from jax.experimental import pallas as pl


def kernel(x, table):
    raise NotImplementedError("write your pallas kernel here")


# speedup vs baseline: 4.4993x; 4.4993x over previous
import functools

import jax
import jax.numpy as jnp
from jax import lax
from jax.experimental import pallas as pl
from jax.experimental.pallas import tpu as pltpu, tpu_sc as plsc

VOCAB = 100000
EMBED_DIM = 64
BATCH = 4096
SEQ = 50

NC = 2
NS = 16
NW = NC * NS
LANES = 16

ROWS = BATCH * SEQ
CHUNK = 128
ROWS_PER_W = ROWS // NW
NCHUNK = ROWS_PER_W // CHUNK
NBUF = 5
NOUTER = NCHUNK // NBUF

assert ROWS % (NW * CHUNK) == 0 and NCHUNK % NBUF == 0


def _emb_kernel(table_hbm, idx_hbm, out_hbm, idx_v, rows_v, gsems, osems):
    wid = lax.axis_index("s") * NC + lax.axis_index("c")
    row_base = wid * ROWS_PER_W

    pltpu.sync_copy(idx_hbm.at[wid], idx_v)

    def fire_gather(c, b):
        pltpu.async_copy(table_hbm.at[idx_v.at[c]], rows_v.at[b], gsems.at[b])

    def wait_gather(b):
        pltpu.make_async_copy(
            table_hbm.at[pl.ds(0, CHUNK)], rows_v.at[b], gsems.at[b]
        ).wait()

    def fire_out(c, b):
        pltpu.async_copy(
            rows_v.at[b], out_hbm.at[pl.ds(row_base + c * CHUNK, CHUNK)],
            osems.at[b],
        )

    def wait_out(b):
        pltpu.make_async_copy(
            table_hbm.at[pl.ds(0, CHUNK)], rows_v.at[b], osems.at[b]
        ).wait()

    for b in range(NBUF):
        fire_gather(b, b)

    zeros16 = jnp.zeros((LANES,), jnp.float32)
    lane_iota = lax.iota(jnp.int32, LANES)

    def body(outer, carry):
        for b in range(NBUF):
            c = outer * NBUF + b
            wait_gather(b)
            for g in range(CHUNK // LANES):
                vec = idx_v[c, pl.ds(g * LANES, LANES)]

                @pl.when(jnp.min(vec) == 0)
                def _zero(vec=vec, g=g, b=b):
                    msk = vec == 0
                    rid = g * LANES + lane_iota
                    for col in range(EMBED_DIM):
                        plsc.store_scatter(
                            rows_v.at[b],
                            [rid, jnp.full((LANES,), col, jnp.int32)],
                            zeros16,
                            mask=msk,
                        )

            fire_out(c, b)

            @pl.when(outer < NOUTER - 1)
            def _next(c=c, b=b):
                wait_out(b)
                fire_gather(c + NBUF, b)

        return carry

    lax.fori_loop(0, NOUTER, body, 0)

    for b in range(NBUF):
        wait_out(b)


@jax.jit
def kernel(x, table):
    idx3d = jnp.reshape(x.astype(jnp.int32), (NW, NCHUNK, CHUNK))
    run = pl.kernel(
        _emb_kernel,
        out_type=jax.ShapeDtypeStruct((ROWS, EMBED_DIM), jnp.float32),
        mesh=plsc.VectorSubcoreMesh(core_axis_name="c", subcore_axis_name="s"),
        compiler_params=pltpu.CompilerParams(
            use_tc_tiling_on_sc=False, needs_layout_passes=False
        ),
        scratch_types=[
            pltpu.VMEM((NCHUNK, CHUNK), jnp.int32),
            pltpu.VMEM((NBUF, CHUNK, EMBED_DIM), jnp.float32),
            pltpu.SemaphoreType.DMA((NBUF,)),
            pltpu.SemaphoreType.DMA((NBUF,)),
        ],
    )
    out = run(table, idx3d)
    return out.reshape(BATCH, SEQ, EMBED_DIM)


# speedup vs baseline: 4.6033x; 1.0231x over previous
import jax
import jax.numpy as jnp
from jax import lax
from jax.experimental import pallas as pl
from jax.experimental.pallas import tpu as pltpu, tpu_sc as plsc

VOCAB = 100000
EMBED_DIM = 64
BATCH = 4096
SEQ = 50

NC = 2
NS = 16
NW = NC * NS
LANES = 16

ROWS = BATCH * SEQ
CHUNK = 128
ROWS_PER_W = ROWS // NW
NCHUNK = ROWS_PER_W // CHUNK
G = 5
GROUP = G * CHUNK
NG = NCHUNK // G
NBUF = 2
NOUTER = NG // NBUF

assert ROWS % (NW * CHUNK) == 0 and NCHUNK % G == 0 and NG % NBUF == 0


def _emb_kernel(table_hbm, idx_hbm, out_hbm, idx_v, rows_v, gsems, osems):
    wid = lax.axis_index("s") * NC + lax.axis_index("c")
    row_base = wid * ROWS_PER_W

    pltpu.sync_copy(idx_hbm.at[wid], idx_v)

    def fire_gathers(g, b):
        for j in range(G):
            pltpu.async_copy(
                table_hbm.at[idx_v.at[g * G + j]],
                rows_v.at[b, pl.ds(j * CHUNK, CHUNK)],
                gsems.at[b],
            )

    def drain_gathers(b):
        for _ in range(G):
            pltpu.make_async_copy(
                table_hbm.at[pl.ds(0, CHUNK)],
                rows_v.at[b, pl.ds(0, CHUNK)],
                gsems.at[b],
            ).wait()

    def fire_out(g, b):
        pltpu.async_copy(
            rows_v.at[b], out_hbm.at[pl.ds(row_base + g * GROUP, GROUP)],
            osems.at[b],
        )

    def wait_out(b):
        pltpu.make_async_copy(
            table_hbm.at[pl.ds(0, GROUP)], rows_v.at[b], osems.at[b]
        ).wait()

    zeros16 = jnp.zeros((LANES,), jnp.float32)
    lane_iota = lax.iota(jnp.int32, LANES)

    def mask_pass(g, b):
        for gg in range(GROUP // LANES):
            vec = idx_v[g * G + gg // (CHUNK // LANES),
                        pl.ds((gg % (CHUNK // LANES)) * LANES, LANES)]

            @pl.when(jnp.min(vec) == 0)
            def _zero(vec=vec, gg=gg, b=b):
                msk = vec == 0
                rid = gg * LANES + lane_iota

                def zcol(col, carry):
                    plsc.store_scatter(
                        rows_v.at[b],
                        [rid, jnp.full((LANES,), 0, jnp.int32) + col],
                        zeros16,
                        mask=msk,
                    )
                    return carry

                lax.fori_loop(0, EMBED_DIM, zcol, 0)

    for b in range(NBUF):
        fire_gathers(b, b)

    def body(outer, carry):
        for b in range(NBUF):
            g = outer * NBUF + b
            drain_gathers(b)
            mask_pass(g, b)
            fire_out(g, b)

            @pl.when(outer < NOUTER - 1)
            def _next(g=g, b=b):
                wait_out(b)
                fire_gathers(g + NBUF, b)

        return carry

    lax.fori_loop(0, NOUTER, body, 0)

    for b in range(NBUF):
        wait_out(b)


@jax.jit
def kernel(x, table):
    idx3d = jnp.reshape(x.astype(jnp.int32), (NW, NCHUNK, CHUNK))
    run = pl.kernel(
        _emb_kernel,
        out_type=jax.ShapeDtypeStruct((ROWS, EMBED_DIM), jnp.float32),
        mesh=plsc.VectorSubcoreMesh(core_axis_name="c", subcore_axis_name="s"),
        compiler_params=pltpu.CompilerParams(
            use_tc_tiling_on_sc=False, needs_layout_passes=False
        ),
        scratch_types=[
            pltpu.VMEM((NCHUNK, CHUNK), jnp.int32),
            pltpu.VMEM((NBUF, GROUP, EMBED_DIM), jnp.float32),
            pltpu.SemaphoreType.DMA((NBUF,)),
            pltpu.SemaphoreType.DMA((NBUF,)),
        ],
    )
    out = run(table, idx3d)
    return out.reshape(BATCH, SEQ, EMBED_DIM)
